# baseline (speedup 1.0000x reference)
import functools

import jax
import jax.numpy as jnp
from jax import lax
from jax.experimental import pallas as pl
from jax.experimental.pallas import tpu as pltpu
from jax.experimental.pallas import tpu_sc as plsc

N = 10000
D = 256
HID = 256
C = 64
E = 160000

NC, NS = 2, 16
CHUNK = 125
CH = 128
PADN = N
HHALF = D // 2
TSW = 128

DEG_CH = E // NS // CHUNK
P1_PADE = 10240
P1_PH = 2
P1_CH = P1_PADE // P1_PH // CH
P2_PADE = 5120
P2_CH = P2_PADE // CH

NPAD = 10240
STRIPE = NPAD // NS

_MESH = plsc.VectorSubcoreMesh(
    core_axis_name="c", subcore_axis_name="s", num_cores=NC, num_subcores=NS)

BM = 1000


def _deg_body(col_hbm, ones_hbm, out_hbm, acc, col_v, ones_v):
    c = lax.axis_index("c")
    s = lax.axis_index("s")
    pltpu.sync_copy(ones_hbm.at[pl.ds(s * STRIPE, STRIPE)],
                    acc.at[pl.ds(s * STRIPE, STRIPE)])
    pltpu.sync_copy(ones_hbm.at[pl.ds(0, 128)], ones_v)
    pltpu.sync_copy(col_hbm.at[c, s], col_v)
    plsc.subcore_barrier()

    def body(j, carry):
        pltpu.sync_copy(ones_v.at[pl.ds(0, CHUNK)], acc.at[col_v.at[j]],
                        add=True)
        return carry

    lax.fori_loop(0, DEG_CH, body, 0)
    plsc.subcore_barrier()
    pltpu.sync_copy(acc.at[pl.ds(s * STRIPE, STRIPE)],
                    out_hbm.at[c, 0, pl.ds(s * STRIPE, STRIPE)])


_deg_call = pl.kernel(
    _deg_body,
    out_type=jax.ShapeDtypeStruct((NC, 1, NPAD), jnp.float32),
    mesh=_MESH,
    scratch_types=[
        pltpu.VMEM_SHARED((NPAD,), jnp.float32),
        pltpu.VMEM((DEG_CH, CHUNK), jnp.int32),
        pltpu.VMEM((128,), jnp.float32),
    ],
)


def _prop1_body(xs_hbm, row_hbm, col_hbm, out_hbm,
                acc, row_v, col_v, buf0):
    c = lax.axis_index("c")
    s = lax.axis_index("s")
    pltpu.sync_copy(xs_hbm.at[c, pl.ds(s * STRIPE, STRIPE)],
                    acc.at[pl.ds(s * STRIPE, STRIPE)])
    xs_c = xs_hbm.at[c]
    for p in range(P1_PH):
        pltpu.sync_copy(row_hbm.at[s, p], row_v)
        pltpu.sync_copy(col_hbm.at[s, p], col_v)
        if p == 0:
            plsc.subcore_barrier()
        def body(j, carry):
            pltpu.sync_copy(xs_c.at[row_v.at[j]], buf0)
            pltpu.sync_copy(buf0, acc.at[col_v.at[j]], add=True)
            return carry

        lax.fori_loop(0, P1_CH, body, 0)
    plsc.subcore_barrier()
    pltpu.sync_copy(acc.at[pl.ds(s * STRIPE, STRIPE)],
                    out_hbm.at[c, pl.ds(s * STRIPE, STRIPE)])


_prop1_call = pl.kernel(
    _prop1_body,
    out_type=jax.ShapeDtypeStruct((NC, NPAD, HHALF), jnp.float32),
    mesh=_MESH,
    scratch_types=[
        pltpu.VMEM_SHARED((NPAD, HHALF), jnp.float32),
        pltpu.VMEM((P1_CH, CH), jnp.int32),
        pltpu.VMEM((P1_CH, CH), jnp.int32),
        pltpu.VMEM((CH, HHALF), jnp.float32),
    ],
)


def _prop2_body(ts_hbm, th_hbm, row_hbm, col_hbm, out_hbm,
                acc, row_v, col_v, buf0):
    c = lax.axis_index("c")
    s = lax.axis_index("s")
    pltpu.sync_copy(th_hbm.at[pl.ds(s * STRIPE, STRIPE)],
                    acc.at[pl.ds(s * STRIPE, STRIPE)])
    pltpu.sync_copy(row_hbm.at[c, s], row_v)
    pltpu.sync_copy(col_hbm.at[c, s], col_v)
    plsc.subcore_barrier()

    def body(j, carry):
        pltpu.sync_copy(ts_hbm.at[row_v.at[j]], buf0)
        pltpu.sync_copy(buf0, acc.at[col_v.at[j]], add=True)
        return carry

    lax.fori_loop(0, P2_CH, body, 0)
    plsc.subcore_barrier()
    pltpu.sync_copy(acc.at[pl.ds(s * STRIPE, STRIPE)],
                    out_hbm.at[c, pl.ds(s * STRIPE, STRIPE)])


_prop2_call = pl.kernel(
    _prop2_body,
    out_type=jax.ShapeDtypeStruct((NC, NPAD, TSW), jnp.float32),
    mesh=_MESH,
    scratch_types=[
        pltpu.VMEM_SHARED((NPAD, TSW), jnp.float32),
        pltpu.VMEM((P2_CH, CH), jnp.int32),
        pltpu.VMEM((P2_CH, CH), jnp.int32),
        pltpu.VMEM((CH, TSW), jnp.float32),
    ],
)


def _scale_body(x_ref, deg_ref, xs_ref, dinv_ref):
    dinv = lax.rsqrt(deg_ref[...])
    dinv_ref[...] = dinv
    xsb = x_ref[...] * dinv
    xs_ref[0] = xsb[:, :HHALF]
    xs_ref[1] = xsb[:, HHALF:]


_scale_call = pl.pallas_call(
    _scale_body,
    grid=(N // BM,),
    in_specs=[
        pl.BlockSpec((BM, D), lambda i: (i, 0)),
        pl.BlockSpec((BM, 1), lambda i: (i, 0)),
    ],
    out_specs=[
        pl.BlockSpec((NC, BM, HHALF), lambda i: (0, i, 0)),
        pl.BlockSpec((BM, 1), lambda i: (i, 0)),
    ],
    out_shape=[
        jax.ShapeDtypeStruct((NC, NPAD, HHALF), jnp.float32),
        jax.ShapeDtypeStruct((NPAD, 1), jnp.float32),
    ],
)


def _mid_body(p1_ref, dinv_ref, w1_ref, b1_ref, w2_ref, ts_ref, th_ref):
    dinv = dinv_ref[...]
    h0 = p1_ref[0] * dinv
    h1 = p1_ref[1] * dinv
    w1 = w1_ref[...]
    a = lax.dot_general(h0, w1[:, :HHALF], (((1,), (1,)), ((), ())),
                        preferred_element_type=jnp.float32)
    a = a + lax.dot_general(h1, w1[:, HHALF:], (((1,), (1,)), ((), ())),
                            preferred_element_type=jnp.float32)
    a = jnp.maximum(a + b1_ref[...], 0.0)
    t = lax.dot_general(a, w2_ref[...], (((1,), (1,)), ((), ())),
                        preferred_element_type=jnp.float32)
    ts = jnp.concatenate(
        [t * dinv, jnp.zeros((t.shape[0], TSW - C), jnp.float32)], axis=1)
    ts_ref[...] = ts
    th_ref[...] = 0.5 * ts


_mid_call = pl.pallas_call(
    _mid_body,
    grid=(N // BM,),
    in_specs=[
        pl.BlockSpec((NC, BM, HHALF), lambda i: (0, i, 0)),
        pl.BlockSpec((BM, 1), lambda i: (i, 0)),
        pl.BlockSpec((HID, D), lambda i: (0, 0)),
        pl.BlockSpec((1, HID), lambda i: (0, 0)),
        pl.BlockSpec((C, HID), lambda i: (0, 0)),
    ],
    out_specs=[
        pl.BlockSpec((BM, TSW), lambda i: (i, 0)),
        pl.BlockSpec((BM, TSW), lambda i: (i, 0)),
    ],
    out_shape=[
        jax.ShapeDtypeStruct((NPAD, TSW), jnp.float32),
        jax.ShapeDtypeStruct((NPAD, TSW), jnp.float32),
    ],
)


def _final_body(p2y_ref, p2z_ref, dy_ref, dz_ref, b2_ref,
                lsy_ref, dist_ref, lsz_ref):
    b2 = b2_ref[...]
    y = (p2y_ref[0] + p2y_ref[1])[:, :C] * dy_ref[...] + b2
    z = (p2z_ref[0] + p2z_ref[1])[:, :C] * dz_ref[...] + b2

    def logsm(v):
        m = jnp.max(v, axis=1, keepdims=True)
        return v - m - jnp.log(jnp.sum(jnp.exp(v - m), axis=1, keepdims=True))

    lsy_ref[...] = logsm(y)
    lsz_ref[...] = logsm(z)
    num = jnp.sum(y * z, axis=1, keepdims=True)
    den = jnp.maximum(
        jnp.sqrt(jnp.sum(y * y, axis=1, keepdims=True)
                 * jnp.sum(z * z, axis=1, keepdims=True)), 1e-8)
    dist_ref[...] = 1.0 - num / den


_final_call = pl.pallas_call(
    _final_body,
    grid=(N // BM,),
    in_specs=[
        pl.BlockSpec((NC, BM, TSW), lambda i: (0, i, 0)),
        pl.BlockSpec((NC, BM, TSW), lambda i: (0, i, 0)),
        pl.BlockSpec((BM, 1), lambda i: (i, 0)),
        pl.BlockSpec((BM, 1), lambda i: (i, 0)),
        pl.BlockSpec((1, C), lambda i: (0, 0)),
    ],
    out_specs=[
        pl.BlockSpec((BM, C), lambda i: (i, 0)),
        pl.BlockSpec((BM, 1), lambda i: (i, 0)),
        pl.BlockSpec((BM, C), lambda i: (i, 0)),
    ],
    out_shape=[
        jax.ShapeDtypeStruct((N, C), jnp.float32),
        jax.ShapeDtypeStruct((N, 1), jnp.float32),
        jax.ShapeDtypeStruct((N, C), jnp.float32),
    ],
)


def _pad_tiles(a, pade):
    pad = pade - a.shape[-1]
    cfg = [(0, 0)] * (a.ndim - 1) + [(0, pad)]
    return jnp.pad(a, cfg, constant_values=PADN)


def _one_graph(x, edge_index, deg, W1, b1_2d, W2):
    row, col = edge_index[0], edge_index[1]
    row1 = _pad_tiles(row.reshape(NS, E // NS), P1_PADE)
    row1 = row1.reshape(NS, P1_PH, P1_CH, CH)
    col1 = _pad_tiles(col.reshape(NS, E // NS), P1_PADE)
    col1 = col1.reshape(NS, P1_PH, P1_CH, CH)
    row2 = _pad_tiles(row.reshape(NC, NS, E // NC // NS), P2_PADE)
    row2 = row2.reshape(NC, NS, P2_CH, CH)
    col2 = _pad_tiles(col.reshape(NC, NS, E // NC // NS), P2_PADE)
    col2 = col2.reshape(NC, NS, P2_CH, CH)
    xs, dinv = _scale_call(x, deg)
    p1 = _prop1_call(xs, row1, col1)
    ts, th = _mid_call(p1, dinv, W1, b1_2d, W2)
    p2 = _prop2_call(ts, th, row2, col2)
    return p2, dinv


def kernel(x, edge_index, x_trans, edge_index_trans, W1, b1, W2, b2):
    ones = jnp.ones((NPAD,), jnp.float32)
    cols2 = jnp.stack([edge_index[1], edge_index_trans[1]])
    cols2 = cols2.reshape(NC, NS, DEG_CH, CHUNK)
    deg2 = _deg_call(cols2, ones)
    deg_y = deg2[0, 0, :, None]
    deg_z = deg2[1, 0, :, None]

    b1_2d = b1.reshape(1, HID)
    p2y, dinv_y = _one_graph(x, edge_index, deg_y, W1, b1_2d, W2)
    p2z, dinv_z = _one_graph(x_trans, edge_index_trans, deg_z, W1, b1_2d, W2)

    ls_y, dist, ls_z = _final_call(p2y, p2z, dinv_y, dinv_z,
                                   b2.reshape(1, C))
    dist = dist.reshape(N)
    return (ls_y, dist, ls_z, ls_y, ls_y)


# speedup vs baseline: 2.0950x; 2.0950x over previous
import functools

import jax
import jax.numpy as jnp
from jax import lax
from jax.experimental import pallas as pl
from jax.experimental.pallas import tpu as pltpu
from jax.experimental.pallas import tpu_sc as plsc

N = 10000
D = 256
HID = 256
C = 64
E = 160000

NC, NS = 2, 16
CHUNK = 125
HHALF = D // 2
TSW = 128

DEG_CH = E // NS // CHUNK
P1_PH = 2
P1_CH = E // NS // P1_PH // CHUNK
P2_CH = E // NC // NS // CHUNK

NPAD = 10240
STRIPE = NPAD // NS

_MESH = plsc.VectorSubcoreMesh(
    core_axis_name="c", subcore_axis_name="s", num_cores=NC, num_subcores=NS)

BM = 1000


def _deg_body(col_hbm, ones_hbm, out_hbm, acc, col_v, ones_v):
    c = lax.axis_index("c")
    s = lax.axis_index("s")
    pltpu.sync_copy(ones_hbm.at[pl.ds(s * STRIPE, STRIPE)],
                    acc.at[pl.ds(s * STRIPE, STRIPE)])
    pltpu.sync_copy(ones_hbm.at[pl.ds(0, 128)], ones_v)
    pltpu.sync_copy(col_hbm.at[c, s], col_v)
    plsc.subcore_barrier()

    def body(j, carry):
        pltpu.sync_copy(ones_v.at[pl.ds(0, CHUNK)], acc.at[col_v.at[j]],
                        add=True)
        return carry

    lax.fori_loop(0, DEG_CH, body, 0)
    plsc.subcore_barrier()
    pltpu.sync_copy(acc.at[pl.ds(s * STRIPE, STRIPE)],
                    out_hbm.at[c, 0, pl.ds(s * STRIPE, STRIPE)])


_deg_call = pl.kernel(
    _deg_body,
    out_type=jax.ShapeDtypeStruct((NC, 1, NPAD), jnp.float32),
    mesh=_MESH,
    scratch_types=[
        pltpu.VMEM_SHARED((NPAD,), jnp.float32),
        pltpu.VMEM((DEG_CH, CHUNK), jnp.int32),
        pltpu.VMEM((128,), jnp.float32),
    ],
)


def _prop1_body(xs_hbm, row_hbm, col_hbm, out_hbm,
                acc, row_v, col_v, buf0, buf1, sem0, sem1):
    c = lax.axis_index("c")
    s = lax.axis_index("s")
    pltpu.sync_copy(xs_hbm.at[c, pl.ds(s * STRIPE, STRIPE)],
                    acc.at[pl.ds(s * STRIPE, STRIPE)])
    xs_c = xs_hbm.at[c]
    for p in range(P1_PH):
        pltpu.sync_copy(row_hbm.at[s, p], row_v)
        pltpu.sync_copy(col_hbm.at[s, p], col_v)
        if p == 0:
            plsc.subcore_barrier()
        def body(i, carry):
            j0 = 2 * i
            d0 = pltpu.async_copy(xs_c.at[row_v.at[j0]], buf0, sem0)
            d1 = pltpu.async_copy(xs_c.at[row_v.at[j0 + 1]], buf1, sem1)
            d0.wait()
            pltpu.sync_copy(buf0, acc.at[col_v.at[j0]], add=True)
            d1.wait()
            pltpu.sync_copy(buf1, acc.at[col_v.at[j0 + 1]], add=True)
            return carry

        lax.fori_loop(0, P1_CH // 2, body, 0)
    plsc.subcore_barrier()
    pltpu.sync_copy(acc.at[pl.ds(s * STRIPE, STRIPE)],
                    out_hbm.at[c, pl.ds(s * STRIPE, STRIPE)])


_prop1_call = pl.kernel(
    _prop1_body,
    out_type=jax.ShapeDtypeStruct((NC, NPAD, HHALF), jnp.float32),
    mesh=_MESH,
    scratch_types=[
        pltpu.VMEM_SHARED((NPAD, HHALF), jnp.float32),
        pltpu.VMEM((P1_CH, CHUNK), jnp.int32),
        pltpu.VMEM((P1_CH, CHUNK), jnp.int32),
        pltpu.VMEM((CHUNK, HHALF), jnp.float32),
        pltpu.VMEM((CHUNK, HHALF), jnp.float32),
        pltpu.SemaphoreType.DMA,
        pltpu.SemaphoreType.DMA,
    ],
)


def _prop2_body(ts_hbm, th_hbm, row_hbm, col_hbm, out_hbm,
                acc, row_v, col_v, buf0, buf1, sem0, sem1):
    c = lax.axis_index("c")
    s = lax.axis_index("s")
    pltpu.sync_copy(th_hbm.at[pl.ds(s * STRIPE, STRIPE)],
                    acc.at[pl.ds(s * STRIPE, STRIPE)])
    pltpu.sync_copy(row_hbm.at[c, s], row_v)
    pltpu.sync_copy(col_hbm.at[c, s], col_v)
    plsc.subcore_barrier()

    def body(i, carry):
        j0 = 2 * i
        d0 = pltpu.async_copy(ts_hbm.at[row_v.at[j0]], buf0, sem0)
        d1 = pltpu.async_copy(ts_hbm.at[row_v.at[j0 + 1]], buf1, sem1)
        d0.wait()
        pltpu.sync_copy(buf0, acc.at[col_v.at[j0]], add=True)
        d1.wait()
        pltpu.sync_copy(buf1, acc.at[col_v.at[j0 + 1]], add=True)
        return carry

    lax.fori_loop(0, P2_CH // 2, body, 0)
    plsc.subcore_barrier()
    pltpu.sync_copy(acc.at[pl.ds(s * STRIPE, STRIPE)],
                    out_hbm.at[c, pl.ds(s * STRIPE, STRIPE)])


_prop2_call = pl.kernel(
    _prop2_body,
    out_type=jax.ShapeDtypeStruct((NC, NPAD, TSW), jnp.float32),
    mesh=_MESH,
    scratch_types=[
        pltpu.VMEM_SHARED((NPAD, TSW), jnp.float32),
        pltpu.VMEM((P2_CH, CHUNK), jnp.int32),
        pltpu.VMEM((P2_CH, CHUNK), jnp.int32),
        pltpu.VMEM((CHUNK, TSW), jnp.float32),
        pltpu.VMEM((CHUNK, TSW), jnp.float32),
        pltpu.SemaphoreType.DMA,
        pltpu.SemaphoreType.DMA,
    ],
)


def _scale_body(x_ref, deg_ref, xs_ref, dinv_ref):
    dinv = lax.rsqrt(deg_ref[...])
    dinv_ref[...] = dinv
    xsb = x_ref[...] * dinv
    xs_ref[0] = xsb[:, :HHALF]
    xs_ref[1] = xsb[:, HHALF:]


_scale_call = pl.pallas_call(
    _scale_body,
    grid=(N // BM,),
    in_specs=[
        pl.BlockSpec((BM, D), lambda i: (i, 0)),
        pl.BlockSpec((BM, 1), lambda i: (i, 0)),
    ],
    out_specs=[
        pl.BlockSpec((NC, BM, HHALF), lambda i: (0, i, 0)),
        pl.BlockSpec((BM, 1), lambda i: (i, 0)),
    ],
    out_shape=[
        jax.ShapeDtypeStruct((NC, NPAD, HHALF), jnp.float32),
        jax.ShapeDtypeStruct((NPAD, 1), jnp.float32),
    ],
)


def _mid_body(p1_ref, dinv_ref, w1_ref, b1_ref, w2_ref, ts_ref, th_ref):
    dinv = dinv_ref[...]
    h0 = p1_ref[0] * dinv
    h1 = p1_ref[1] * dinv
    w1 = w1_ref[...]
    a = lax.dot_general(h0, w1[:, :HHALF], (((1,), (1,)), ((), ())),
                        preferred_element_type=jnp.float32)
    a = a + lax.dot_general(h1, w1[:, HHALF:], (((1,), (1,)), ((), ())),
                            preferred_element_type=jnp.float32)
    a = jnp.maximum(a + b1_ref[...], 0.0)
    t = lax.dot_general(a, w2_ref[...], (((1,), (1,)), ((), ())),
                        preferred_element_type=jnp.float32)
    ts = jnp.concatenate(
        [t * dinv, jnp.zeros((t.shape[0], TSW - C), jnp.float32)], axis=1)
    ts_ref[...] = ts
    th_ref[...] = 0.5 * ts


_mid_call = pl.pallas_call(
    _mid_body,
    grid=(N // BM,),
    in_specs=[
        pl.BlockSpec((NC, BM, HHALF), lambda i: (0, i, 0)),
        pl.BlockSpec((BM, 1), lambda i: (i, 0)),
        pl.BlockSpec((HID, D), lambda i: (0, 0)),
        pl.BlockSpec((1, HID), lambda i: (0, 0)),
        pl.BlockSpec((C, HID), lambda i: (0, 0)),
    ],
    out_specs=[
        pl.BlockSpec((BM, TSW), lambda i: (i, 0)),
        pl.BlockSpec((BM, TSW), lambda i: (i, 0)),
    ],
    out_shape=[
        jax.ShapeDtypeStruct((NPAD, TSW), jnp.float32),
        jax.ShapeDtypeStruct((NPAD, TSW), jnp.float32),
    ],
)


def _final_body(p2y_ref, p2z_ref, dy_ref, dz_ref, b2_ref,
                lsy_ref, dist_ref, lsz_ref):
    b2 = b2_ref[...]
    y = (p2y_ref[0] + p2y_ref[1])[:, :C] * dy_ref[...] + b2
    z = (p2z_ref[0] + p2z_ref[1])[:, :C] * dz_ref[...] + b2

    def logsm(v):
        m = jnp.max(v, axis=1, keepdims=True)
        return v - m - jnp.log(jnp.sum(jnp.exp(v - m), axis=1, keepdims=True))

    lsy_ref[...] = logsm(y)
    lsz_ref[...] = logsm(z)
    num = jnp.sum(y * z, axis=1, keepdims=True)
    den = jnp.maximum(
        jnp.sqrt(jnp.sum(y * y, axis=1, keepdims=True)
                 * jnp.sum(z * z, axis=1, keepdims=True)), 1e-8)
    dist_ref[...] = 1.0 - num / den


_final_call = pl.pallas_call(
    _final_body,
    grid=(N // BM,),
    in_specs=[
        pl.BlockSpec((NC, BM, TSW), lambda i: (0, i, 0)),
        pl.BlockSpec((NC, BM, TSW), lambda i: (0, i, 0)),
        pl.BlockSpec((BM, 1), lambda i: (i, 0)),
        pl.BlockSpec((BM, 1), lambda i: (i, 0)),
        pl.BlockSpec((1, C), lambda i: (0, 0)),
    ],
    out_specs=[
        pl.BlockSpec((BM, C), lambda i: (i, 0)),
        pl.BlockSpec((BM, 1), lambda i: (i, 0)),
        pl.BlockSpec((BM, C), lambda i: (i, 0)),
    ],
    out_shape=[
        jax.ShapeDtypeStruct((N, C), jnp.float32),
        jax.ShapeDtypeStruct((N, 1), jnp.float32),
        jax.ShapeDtypeStruct((N, C), jnp.float32),
    ],
)


def _one_graph(x, edge_index, deg, W1, b1_2d, W2):
    row, col = edge_index[0], edge_index[1]
    row1 = row.reshape(NS, P1_PH, P1_CH, CHUNK)
    col1 = col.reshape(NS, P1_PH, P1_CH, CHUNK)
    row2 = row.reshape(NC, NS, P2_CH, CHUNK)
    col2 = col.reshape(NC, NS, P2_CH, CHUNK)
    xs, dinv = _scale_call(x, deg)
    p1 = _prop1_call(xs, row1, col1)
    ts, th = _mid_call(p1, dinv, W1, b1_2d, W2)
    p2 = _prop2_call(ts, th, row2, col2)
    return p2, dinv


def kernel(x, edge_index, x_trans, edge_index_trans, W1, b1, W2, b2):
    ones = jnp.ones((NPAD,), jnp.float32)
    cols2 = jnp.stack([edge_index[1], edge_index_trans[1]])
    cols2 = cols2.reshape(NC, NS, DEG_CH, CHUNK)
    deg2 = _deg_call(cols2, ones)
    deg_y = deg2[0, 0, :, None]
    deg_z = deg2[1, 0, :, None]

    b1_2d = b1.reshape(1, HID)
    p2y, dinv_y = _one_graph(x, edge_index, deg_y, W1, b1_2d, W2)
    p2z, dinv_z = _one_graph(x_trans, edge_index_trans, deg_z, W1, b1_2d, W2)

    ls_y, dist, ls_z = _final_call(p2y, p2z, dinv_y, dinv_z,
                                   b2.reshape(1, C))
    dist = dist.reshape(N)
    return (ls_y, dist, ls_z, ls_y, ls_y)


# speedup vs baseline: 2.4657x; 1.1769x over previous
import functools

import jax
import jax.numpy as jnp
from jax import lax
from jax.experimental import pallas as pl
from jax.experimental.pallas import tpu as pltpu
from jax.experimental.pallas import tpu_sc as plsc

N = 10000
D = 256
HID = 256
C = 64
E = 160000

NC, NS = 2, 16
CHUNK = 125
HHALF = D // 2
TSW = 128

DEG_CH = E // NS // CHUNK
P1_PH = 2
P1_CH = E // NS // P1_PH // CHUNK
P2_CH = E // NC // NS // CHUNK

NPAD = 10240
STRIPE = NPAD // NS

_MESH = plsc.VectorSubcoreMesh(
    core_axis_name="c", subcore_axis_name="s", num_cores=NC, num_subcores=NS)

BM = 1000


GROUP = 8


def _pipe_group(src, row_v, col_v, acc, buf0, buf1, sem0, sem1, nch):
    bufs = (buf0, buf1)
    sems = (sem0, sem1)

    def body(i, carry):
        base = i * GROUP
        ds = [pltpu.async_copy(src.at[row_v.at[base]], bufs[0], sems[0]),
              pltpu.async_copy(src.at[row_v.at[base + 1]], bufs[1], sems[1])]
        for k in range(GROUP):
            b = k % 2
            ds[b].wait()
            pltpu.sync_copy(bufs[b], acc.at[col_v.at[base + k]], add=True)
            if k + 2 < GROUP:
                ds[b] = pltpu.async_copy(src.at[row_v.at[base + k + 2]],
                                         bufs[b], sems[b])
        return carry

    lax.fori_loop(0, nch // GROUP, body, 0)


def _deg_body(col_hbm, ones_hbm, out_hbm, acc, col_v, ones_v):
    c = lax.axis_index("c")
    s = lax.axis_index("s")
    pltpu.sync_copy(ones_hbm.at[pl.ds(s * STRIPE, STRIPE)],
                    acc.at[pl.ds(s * STRIPE, STRIPE)])
    pltpu.sync_copy(ones_hbm.at[pl.ds(0, 128)], ones_v)
    pltpu.sync_copy(col_hbm.at[c, s], col_v)
    plsc.subcore_barrier()

    def body(j, carry):
        pltpu.sync_copy(ones_v.at[pl.ds(0, CHUNK)], acc.at[col_v.at[j]],
                        add=True)
        return carry

    lax.fori_loop(0, DEG_CH, body, 0)
    plsc.subcore_barrier()
    pltpu.sync_copy(acc.at[pl.ds(s * STRIPE, STRIPE)],
                    out_hbm.at[c, 0, pl.ds(s * STRIPE, STRIPE)])


_deg_call = pl.kernel(
    _deg_body,
    out_type=jax.ShapeDtypeStruct((NC, 1, NPAD), jnp.float32),
    mesh=_MESH,
    scratch_types=[
        pltpu.VMEM_SHARED((NPAD,), jnp.float32),
        pltpu.VMEM((DEG_CH, CHUNK), jnp.int32),
        pltpu.VMEM((128,), jnp.float32),
    ],
)


def _prop1_body(xs_hbm, row_hbm, col_hbm, out_hbm,
                acc, row_v, col_v, buf0, buf1, sem0, sem1):
    c = lax.axis_index("c")
    s = lax.axis_index("s")
    pltpu.sync_copy(xs_hbm.at[c, pl.ds(s * STRIPE, STRIPE)],
                    acc.at[pl.ds(s * STRIPE, STRIPE)])
    xs_c = xs_hbm.at[c]
    for p in range(P1_PH):
        pltpu.sync_copy(row_hbm.at[s, p], row_v)
        pltpu.sync_copy(col_hbm.at[s, p], col_v)
        if p == 0:
            plsc.subcore_barrier()
        _pipe_group(xs_c, row_v, col_v, acc, buf0, buf1, sem0, sem1,
                    P1_CH)
    plsc.subcore_barrier()
    pltpu.sync_copy(acc.at[pl.ds(s * STRIPE, STRIPE)],
                    out_hbm.at[c, pl.ds(s * STRIPE, STRIPE)])


_prop1_call = pl.kernel(
    _prop1_body,
    out_type=jax.ShapeDtypeStruct((NC, NPAD, HHALF), jnp.float32),
    mesh=_MESH,
    scratch_types=[
        pltpu.VMEM_SHARED((NPAD, HHALF), jnp.float32),
        pltpu.VMEM((P1_CH, CHUNK), jnp.int32),
        pltpu.VMEM((P1_CH, CHUNK), jnp.int32),
        pltpu.VMEM((CHUNK, HHALF), jnp.float32),
        pltpu.VMEM((CHUNK, HHALF), jnp.float32),
        pltpu.SemaphoreType.DMA,
        pltpu.SemaphoreType.DMA,
    ],
)


def _prop2_body(ts_hbm, th_hbm, row_hbm, col_hbm, out_hbm,
                acc, row_v, col_v, buf0, buf1, sem0, sem1):
    c = lax.axis_index("c")
    s = lax.axis_index("s")
    pltpu.sync_copy(th_hbm.at[pl.ds(s * STRIPE, STRIPE)],
                    acc.at[pl.ds(s * STRIPE, STRIPE)])
    pltpu.sync_copy(row_hbm.at[c, s], row_v)
    pltpu.sync_copy(col_hbm.at[c, s], col_v)
    plsc.subcore_barrier()

    _pipe_group(ts_hbm, row_v, col_v, acc, buf0, buf1, sem0, sem1, P2_CH)
    plsc.subcore_barrier()
    pltpu.sync_copy(acc.at[pl.ds(s * STRIPE, STRIPE)],
                    out_hbm.at[c, pl.ds(s * STRIPE, STRIPE)])


_prop2_call = pl.kernel(
    _prop2_body,
    out_type=jax.ShapeDtypeStruct((NC, NPAD, TSW), jnp.float32),
    mesh=_MESH,
    scratch_types=[
        pltpu.VMEM_SHARED((NPAD, TSW), jnp.float32),
        pltpu.VMEM((P2_CH, CHUNK), jnp.int32),
        pltpu.VMEM((P2_CH, CHUNK), jnp.int32),
        pltpu.VMEM((CHUNK, TSW), jnp.float32),
        pltpu.VMEM((CHUNK, TSW), jnp.float32),
        pltpu.SemaphoreType.DMA,
        pltpu.SemaphoreType.DMA,
    ],
)


def _scale_body(x_ref, deg_ref, xs_ref, dinv_ref):
    dinv = lax.rsqrt(deg_ref[...])
    dinv_ref[...] = dinv
    xsb = x_ref[...] * dinv
    xs_ref[0] = xsb[:, :HHALF]
    xs_ref[1] = xsb[:, HHALF:]


_scale_call = pl.pallas_call(
    _scale_body,
    grid=(N // BM,),
    in_specs=[
        pl.BlockSpec((BM, D), lambda i: (i, 0)),
        pl.BlockSpec((BM, 1), lambda i: (i, 0)),
    ],
    out_specs=[
        pl.BlockSpec((NC, BM, HHALF), lambda i: (0, i, 0)),
        pl.BlockSpec((BM, 1), lambda i: (i, 0)),
    ],
    out_shape=[
        jax.ShapeDtypeStruct((NC, NPAD, HHALF), jnp.float32),
        jax.ShapeDtypeStruct((NPAD, 1), jnp.float32),
    ],
)


def _mid_body(p1_ref, dinv_ref, w1_ref, b1_ref, w2_ref, ts_ref, th_ref):
    dinv = dinv_ref[...]
    h0 = p1_ref[0] * dinv
    h1 = p1_ref[1] * dinv
    w1 = w1_ref[...]
    a = lax.dot_general(h0, w1[:, :HHALF], (((1,), (1,)), ((), ())),
                        preferred_element_type=jnp.float32)
    a = a + lax.dot_general(h1, w1[:, HHALF:], (((1,), (1,)), ((), ())),
                            preferred_element_type=jnp.float32)
    a = jnp.maximum(a + b1_ref[...], 0.0)
    t = lax.dot_general(a, w2_ref[...], (((1,), (1,)), ((), ())),
                        preferred_element_type=jnp.float32)
    ts = jnp.concatenate(
        [t * dinv, jnp.zeros((t.shape[0], TSW - C), jnp.float32)], axis=1)
    ts_ref[...] = ts
    th_ref[...] = 0.5 * ts


_mid_call = pl.pallas_call(
    _mid_body,
    grid=(N // BM,),
    in_specs=[
        pl.BlockSpec((NC, BM, HHALF), lambda i: (0, i, 0)),
        pl.BlockSpec((BM, 1), lambda i: (i, 0)),
        pl.BlockSpec((HID, D), lambda i: (0, 0)),
        pl.BlockSpec((1, HID), lambda i: (0, 0)),
        pl.BlockSpec((C, HID), lambda i: (0, 0)),
    ],
    out_specs=[
        pl.BlockSpec((BM, TSW), lambda i: (i, 0)),
        pl.BlockSpec((BM, TSW), lambda i: (i, 0)),
    ],
    out_shape=[
        jax.ShapeDtypeStruct((NPAD, TSW), jnp.float32),
        jax.ShapeDtypeStruct((NPAD, TSW), jnp.float32),
    ],
)


def _final_body(p2y_ref, p2z_ref, dy_ref, dz_ref, b2_ref,
                lsy_ref, dist_ref, lsz_ref):
    b2 = b2_ref[...]
    y = (p2y_ref[0] + p2y_ref[1])[:, :C] * dy_ref[...] + b2
    z = (p2z_ref[0] + p2z_ref[1])[:, :C] * dz_ref[...] + b2

    def logsm(v):
        m = jnp.max(v, axis=1, keepdims=True)
        return v - m - jnp.log(jnp.sum(jnp.exp(v - m), axis=1, keepdims=True))

    lsy_ref[...] = logsm(y)
    lsz_ref[...] = logsm(z)
    num = jnp.sum(y * z, axis=1, keepdims=True)
    den = jnp.maximum(
        jnp.sqrt(jnp.sum(y * y, axis=1, keepdims=True)
                 * jnp.sum(z * z, axis=1, keepdims=True)), 1e-8)
    dist_ref[...] = 1.0 - num / den


_final_call = pl.pallas_call(
    _final_body,
    grid=(N // BM,),
    in_specs=[
        pl.BlockSpec((NC, BM, TSW), lambda i: (0, i, 0)),
        pl.BlockSpec((NC, BM, TSW), lambda i: (0, i, 0)),
        pl.BlockSpec((BM, 1), lambda i: (i, 0)),
        pl.BlockSpec((BM, 1), lambda i: (i, 0)),
        pl.BlockSpec((1, C), lambda i: (0, 0)),
    ],
    out_specs=[
        pl.BlockSpec((BM, C), lambda i: (i, 0)),
        pl.BlockSpec((BM, 1), lambda i: (i, 0)),
        pl.BlockSpec((BM, C), lambda i: (i, 0)),
    ],
    out_shape=[
        jax.ShapeDtypeStruct((N, C), jnp.float32),
        jax.ShapeDtypeStruct((N, 1), jnp.float32),
        jax.ShapeDtypeStruct((N, C), jnp.float32),
    ],
)


def _one_graph(x, edge_index, deg, W1, b1_2d, W2):
    row, col = edge_index[0], edge_index[1]
    row1 = row.reshape(NS, P1_PH, P1_CH, CHUNK)
    col1 = col.reshape(NS, P1_PH, P1_CH, CHUNK)
    row2 = row.reshape(NC, NS, P2_CH, CHUNK)
    col2 = col.reshape(NC, NS, P2_CH, CHUNK)
    xs, dinv = _scale_call(x, deg)
    p1 = _prop1_call(xs, row1, col1)
    ts, th = _mid_call(p1, dinv, W1, b1_2d, W2)
    p2 = _prop2_call(ts, th, row2, col2)
    return p2, dinv


def kernel(x, edge_index, x_trans, edge_index_trans, W1, b1, W2, b2):
    ones = jnp.ones((NPAD,), jnp.float32)
    cols2 = jnp.stack([edge_index[1], edge_index_trans[1]])
    cols2 = cols2.reshape(NC, NS, DEG_CH, CHUNK)
    deg2 = _deg_call(cols2, ones)
    deg_y = deg2[0, 0, :, None]
    deg_z = deg2[1, 0, :, None]

    b1_2d = b1.reshape(1, HID)
    p2y, dinv_y = _one_graph(x, edge_index, deg_y, W1, b1_2d, W2)
    p2z, dinv_z = _one_graph(x_trans, edge_index_trans, deg_z, W1, b1_2d, W2)

    ls_y, dist, ls_z = _final_call(p2y, p2z, dinv_y, dinv_z,
                                   b2.reshape(1, C))
    dist = dist.reshape(N)
    return (ls_y, dist, ls_z, ls_y, ls_y)


# speedup vs baseline: 2.5845x; 1.0482x over previous
import functools

import jax
import jax.numpy as jnp
from jax import lax
from jax.experimental import pallas as pl
from jax.experimental.pallas import tpu as pltpu
from jax.experimental.pallas import tpu_sc as plsc

N = 10000
D = 256
HID = 256
C = 64
E = 160000

NC, NS = 2, 16
CHUNK = 125
HHALF = D // 2
TSW = 128

DEG_CH = E // NS // CHUNK
P1_PH = 2
P1_CH = E // NS // P1_PH // CHUNK
P2_CH = E // NC // NS // CHUNK

NPAD = 10240
STRIPE = NPAD // NS

_MESH = plsc.VectorSubcoreMesh(
    core_axis_name="c", subcore_axis_name="s", num_cores=NC, num_subcores=NS)

BM = 1000


GROUP = 20


def _pipe_group(src, row_v, col_v, acc, buf0, buf1, sem0, sem1, nch):
    bufs = (buf0, buf1)
    sems = (sem0, sem1)

    def body(i, carry):
        base = i * GROUP
        ds = [pltpu.async_copy(src.at[row_v.at[base]], bufs[0], sems[0]),
              pltpu.async_copy(src.at[row_v.at[base + 1]], bufs[1], sems[1])]
        for k in range(GROUP):
            b = k % 2
            ds[b].wait()
            pltpu.sync_copy(bufs[b], acc.at[col_v.at[base + k]], add=True)
            if k + 2 < GROUP:
                ds[b] = pltpu.async_copy(src.at[row_v.at[base + k + 2]],
                                         bufs[b], sems[b])
        return carry

    lax.fori_loop(0, nch // GROUP, body, 0)


def _deg_body(col_hbm, ones_hbm, out_hbm, acc, col_v, ones_v):
    c = lax.axis_index("c")
    s = lax.axis_index("s")
    pltpu.sync_copy(ones_hbm.at[pl.ds(s * STRIPE, STRIPE)],
                    acc.at[pl.ds(s * STRIPE, STRIPE)])
    pltpu.sync_copy(ones_hbm.at[pl.ds(0, 128)], ones_v)
    pltpu.sync_copy(col_hbm.at[c, s], col_v)
    plsc.subcore_barrier()

    def body(j, carry):
        pltpu.sync_copy(ones_v.at[pl.ds(0, CHUNK)], acc.at[col_v.at[j]],
                        add=True)
        return carry

    lax.fori_loop(0, DEG_CH, body, 0)
    plsc.subcore_barrier()
    pltpu.sync_copy(acc.at[pl.ds(s * STRIPE, STRIPE)],
                    out_hbm.at[c, 0, pl.ds(s * STRIPE, STRIPE)])


_deg_call = pl.kernel(
    _deg_body,
    out_type=jax.ShapeDtypeStruct((NC, 1, NPAD), jnp.float32),
    mesh=_MESH,
    scratch_types=[
        pltpu.VMEM_SHARED((NPAD,), jnp.float32),
        pltpu.VMEM((DEG_CH, CHUNK), jnp.int32),
        pltpu.VMEM((128,), jnp.float32),
    ],
)


def _prop1_body(xs_hbm, row_hbm, col_hbm, out_hbm,
                acc, row_v, col_v, buf0, buf1, sem0, sem1):
    c = lax.axis_index("c")
    s = lax.axis_index("s")
    pltpu.sync_copy(xs_hbm.at[c, pl.ds(s * STRIPE, STRIPE)],
                    acc.at[pl.ds(s * STRIPE, STRIPE)])
    xs_c = xs_hbm.at[c]
    for p in range(P1_PH):
        pltpu.sync_copy(row_hbm.at[s, p], row_v)
        pltpu.sync_copy(col_hbm.at[s, p], col_v)
        if p == 0:
            plsc.subcore_barrier()
        _pipe_group(xs_c, row_v, col_v, acc, buf0, buf1, sem0, sem1,
                    P1_CH)
    plsc.subcore_barrier()
    pltpu.sync_copy(acc.at[pl.ds(s * STRIPE, STRIPE)],
                    out_hbm.at[c, pl.ds(s * STRIPE, STRIPE)])


_prop1_call = pl.kernel(
    _prop1_body,
    out_type=jax.ShapeDtypeStruct((NC, NPAD, HHALF), jnp.float32),
    mesh=_MESH,
    scratch_types=[
        pltpu.VMEM_SHARED((NPAD, HHALF), jnp.float32),
        pltpu.VMEM((P1_CH, CHUNK), jnp.int32),
        pltpu.VMEM((P1_CH, CHUNK), jnp.int32),
        pltpu.VMEM((CHUNK, HHALF), jnp.float32),
        pltpu.VMEM((CHUNK, HHALF), jnp.float32),
        pltpu.SemaphoreType.DMA,
        pltpu.SemaphoreType.DMA,
    ],
)


def _prop2_body(ts_hbm, th_hbm, row_hbm, col_hbm, out_hbm,
                acc, row_v, col_v, buf0, buf1, sem0, sem1):
    c = lax.axis_index("c")
    s = lax.axis_index("s")
    pltpu.sync_copy(th_hbm.at[pl.ds(s * STRIPE, STRIPE)],
                    acc.at[pl.ds(s * STRIPE, STRIPE)])
    pltpu.sync_copy(row_hbm.at[c, s], row_v)
    pltpu.sync_copy(col_hbm.at[c, s], col_v)
    plsc.subcore_barrier()

    _pipe_group(ts_hbm, row_v, col_v, acc, buf0, buf1, sem0, sem1, P2_CH)
    plsc.subcore_barrier()
    pltpu.sync_copy(acc.at[pl.ds(s * STRIPE, STRIPE)],
                    out_hbm.at[c, pl.ds(s * STRIPE, STRIPE)])


_prop2_call = pl.kernel(
    _prop2_body,
    out_type=jax.ShapeDtypeStruct((NC, NPAD, TSW), jnp.float32),
    mesh=_MESH,
    scratch_types=[
        pltpu.VMEM_SHARED((NPAD, TSW), jnp.float32),
        pltpu.VMEM((P2_CH, CHUNK), jnp.int32),
        pltpu.VMEM((P2_CH, CHUNK), jnp.int32),
        pltpu.VMEM((CHUNK, TSW), jnp.float32),
        pltpu.VMEM((CHUNK, TSW), jnp.float32),
        pltpu.SemaphoreType.DMA,
        pltpu.SemaphoreType.DMA,
    ],
)


def _scale_body(x_ref, deg_ref, xs_ref, dinv_ref):
    dinv = lax.rsqrt(deg_ref[...])
    dinv_ref[...] = dinv
    xsb = x_ref[...] * dinv
    xs_ref[0] = xsb[:, :HHALF]
    xs_ref[1] = xsb[:, HHALF:]


_scale_call = pl.pallas_call(
    _scale_body,
    grid=(N // BM,),
    in_specs=[
        pl.BlockSpec((BM, D), lambda i: (i, 0)),
        pl.BlockSpec((BM, 1), lambda i: (i, 0)),
    ],
    out_specs=[
        pl.BlockSpec((NC, BM, HHALF), lambda i: (0, i, 0)),
        pl.BlockSpec((BM, 1), lambda i: (i, 0)),
    ],
    out_shape=[
        jax.ShapeDtypeStruct((NC, NPAD, HHALF), jnp.float32),
        jax.ShapeDtypeStruct((NPAD, 1), jnp.float32),
    ],
)


def _mid_body(p1_ref, dinv_ref, w1_ref, b1_ref, w2_ref, ts_ref, th_ref):
    dinv = dinv_ref[...]
    h0 = p1_ref[0] * dinv
    h1 = p1_ref[1] * dinv
    w1 = w1_ref[...]
    a = lax.dot_general(h0, w1[:, :HHALF], (((1,), (1,)), ((), ())),
                        preferred_element_type=jnp.float32)
    a = a + lax.dot_general(h1, w1[:, HHALF:], (((1,), (1,)), ((), ())),
                            preferred_element_type=jnp.float32)
    a = jnp.maximum(a + b1_ref[...], 0.0)
    t = lax.dot_general(a, w2_ref[...], (((1,), (1,)), ((), ())),
                        preferred_element_type=jnp.float32)
    ts = jnp.concatenate(
        [t * dinv, jnp.zeros((t.shape[0], TSW - C), jnp.float32)], axis=1)
    ts_ref[...] = ts
    th_ref[...] = 0.5 * ts


_mid_call = pl.pallas_call(
    _mid_body,
    grid=(N // BM,),
    in_specs=[
        pl.BlockSpec((NC, BM, HHALF), lambda i: (0, i, 0)),
        pl.BlockSpec((BM, 1), lambda i: (i, 0)),
        pl.BlockSpec((HID, D), lambda i: (0, 0)),
        pl.BlockSpec((1, HID), lambda i: (0, 0)),
        pl.BlockSpec((C, HID), lambda i: (0, 0)),
    ],
    out_specs=[
        pl.BlockSpec((BM, TSW), lambda i: (i, 0)),
        pl.BlockSpec((BM, TSW), lambda i: (i, 0)),
    ],
    out_shape=[
        jax.ShapeDtypeStruct((NPAD, TSW), jnp.float32),
        jax.ShapeDtypeStruct((NPAD, TSW), jnp.float32),
    ],
)


def _final_body(p2y_ref, p2z_ref, dy_ref, dz_ref, b2_ref,
                lsy_ref, dist_ref, lsz_ref):
    b2 = b2_ref[...]
    y = (p2y_ref[0] + p2y_ref[1])[:, :C] * dy_ref[...] + b2
    z = (p2z_ref[0] + p2z_ref[1])[:, :C] * dz_ref[...] + b2

    def logsm(v):
        m = jnp.max(v, axis=1, keepdims=True)
        return v - m - jnp.log(jnp.sum(jnp.exp(v - m), axis=1, keepdims=True))

    lsy_ref[...] = logsm(y)
    lsz_ref[...] = logsm(z)
    num = jnp.sum(y * z, axis=1, keepdims=True)
    den = jnp.maximum(
        jnp.sqrt(jnp.sum(y * y, axis=1, keepdims=True)
                 * jnp.sum(z * z, axis=1, keepdims=True)), 1e-8)
    dist_ref[...] = 1.0 - num / den


_final_call = pl.pallas_call(
    _final_body,
    grid=(N // BM,),
    in_specs=[
        pl.BlockSpec((NC, BM, TSW), lambda i: (0, i, 0)),
        pl.BlockSpec((NC, BM, TSW), lambda i: (0, i, 0)),
        pl.BlockSpec((BM, 1), lambda i: (i, 0)),
        pl.BlockSpec((BM, 1), lambda i: (i, 0)),
        pl.BlockSpec((1, C), lambda i: (0, 0)),
    ],
    out_specs=[
        pl.BlockSpec((BM, C), lambda i: (i, 0)),
        pl.BlockSpec((BM, 1), lambda i: (i, 0)),
        pl.BlockSpec((BM, C), lambda i: (i, 0)),
    ],
    out_shape=[
        jax.ShapeDtypeStruct((N, C), jnp.float32),
        jax.ShapeDtypeStruct((N, 1), jnp.float32),
        jax.ShapeDtypeStruct((N, C), jnp.float32),
    ],
)


def _one_graph(x, edge_index, deg, W1, b1_2d, W2):
    row, col = edge_index[0], edge_index[1]
    row1 = row.reshape(NS, P1_PH, P1_CH, CHUNK)
    col1 = col.reshape(NS, P1_PH, P1_CH, CHUNK)
    row2 = row.reshape(NC, NS, P2_CH, CHUNK)
    col2 = col.reshape(NC, NS, P2_CH, CHUNK)
    xs, dinv = _scale_call(x, deg)
    p1 = _prop1_call(xs, row1, col1)
    ts, th = _mid_call(p1, dinv, W1, b1_2d, W2)
    p2 = _prop2_call(ts, th, row2, col2)
    return p2, dinv


def kernel(x, edge_index, x_trans, edge_index_trans, W1, b1, W2, b2):
    ones = jnp.ones((NPAD,), jnp.float32)
    cols2 = jnp.stack([edge_index[1], edge_index_trans[1]])
    cols2 = cols2.reshape(NC, NS, DEG_CH, CHUNK)
    deg2 = _deg_call(cols2, ones)
    deg_y = deg2[0, 0, :, None]
    deg_z = deg2[1, 0, :, None]

    b1_2d = b1.reshape(1, HID)
    p2y, dinv_y = _one_graph(x, edge_index, deg_y, W1, b1_2d, W2)
    p2z, dinv_z = _one_graph(x_trans, edge_index_trans, deg_z, W1, b1_2d, W2)

    ls_y, dist, ls_z = _final_call(p2y, p2z, dinv_y, dinv_z,
                                   b2.reshape(1, C))
    dist = dist.reshape(N)
    return (ls_y, dist, ls_z, ls_y, ls_y)


# speedup vs baseline: 2.5886x; 1.0016x over previous
import functools

import jax
import jax.numpy as jnp
from jax import lax
from jax.experimental import pallas as pl
from jax.experimental.pallas import tpu as pltpu
from jax.experimental.pallas import tpu_sc as plsc

N = 10000
D = 256
HID = 256
C = 64
E = 160000

NC, NS = 2, 16
CHUNK = 125
HHALF = D // 2
TSW = 128

DEG_CH = E // NS // CHUNK
P1_PH = 2
P1_CH = E // NS // P1_PH // CHUNK
P2_CH = E // NC // NS // CHUNK

NPAD = 10240
STRIPE = NPAD // NS

_MESH = plsc.VectorSubcoreMesh(
    core_axis_name="c", subcore_axis_name="s", num_cores=NC, num_subcores=NS)

BM = 1000


GROUP = 40


def _pipe_group(src, row_v, col_v, acc, buf0, buf1, sem0, sem1, nch):
    bufs = (buf0, buf1)
    sems = (sem0, sem1)

    def body(i, carry):
        base = i * GROUP
        ds = [pltpu.async_copy(src.at[row_v.at[base]], bufs[0], sems[0]),
              pltpu.async_copy(src.at[row_v.at[base + 1]], bufs[1], sems[1])]
        for k in range(GROUP):
            b = k % 2
            ds[b].wait()
            pltpu.sync_copy(bufs[b], acc.at[col_v.at[base + k]], add=True)
            if k + 2 < GROUP:
                ds[b] = pltpu.async_copy(src.at[row_v.at[base + k + 2]],
                                         bufs[b], sems[b])
        return carry

    lax.fori_loop(0, nch // GROUP, body, 0)


def _deg_body(col_hbm, ones_hbm, out_hbm, acc, col_v, ones_v):
    c = lax.axis_index("c")
    s = lax.axis_index("s")
    pltpu.sync_copy(ones_hbm.at[pl.ds(s * STRIPE, STRIPE)],
                    acc.at[pl.ds(s * STRIPE, STRIPE)])
    pltpu.sync_copy(ones_hbm.at[pl.ds(0, 128)], ones_v)
    pltpu.sync_copy(col_hbm.at[c, s], col_v)
    plsc.subcore_barrier()

    def body(j, carry):
        pltpu.sync_copy(ones_v.at[pl.ds(0, CHUNK)], acc.at[col_v.at[j]],
                        add=True)
        return carry

    lax.fori_loop(0, DEG_CH, body, 0)
    plsc.subcore_barrier()
    pltpu.sync_copy(acc.at[pl.ds(s * STRIPE, STRIPE)],
                    out_hbm.at[c, 0, pl.ds(s * STRIPE, STRIPE)])


_deg_call = pl.kernel(
    _deg_body,
    out_type=jax.ShapeDtypeStruct((NC, 1, NPAD), jnp.float32),
    mesh=_MESH,
    scratch_types=[
        pltpu.VMEM_SHARED((NPAD,), jnp.float32),
        pltpu.VMEM((DEG_CH, CHUNK), jnp.int32),
        pltpu.VMEM((128,), jnp.float32),
    ],
)


def _prop1_body(xs_hbm, row_hbm, col_hbm, out_hbm,
                acc, row_v, col_v, buf0, buf1, sem0, sem1):
    c = lax.axis_index("c")
    s = lax.axis_index("s")
    pltpu.sync_copy(xs_hbm.at[c, pl.ds(s * STRIPE, STRIPE)],
                    acc.at[pl.ds(s * STRIPE, STRIPE)])
    xs_c = xs_hbm.at[c]
    for p in range(P1_PH):
        pltpu.sync_copy(row_hbm.at[s, p], row_v)
        pltpu.sync_copy(col_hbm.at[s, p], col_v)
        if p == 0:
            plsc.subcore_barrier()
        _pipe_group(xs_c, row_v, col_v, acc, buf0, buf1, sem0, sem1,
                    P1_CH)
    plsc.subcore_barrier()
    pltpu.sync_copy(acc.at[pl.ds(s * STRIPE, STRIPE)],
                    out_hbm.at[c, pl.ds(s * STRIPE, STRIPE)])


_prop1_call = pl.kernel(
    _prop1_body,
    out_type=jax.ShapeDtypeStruct((NC, NPAD, HHALF), jnp.float32),
    mesh=_MESH,
    scratch_types=[
        pltpu.VMEM_SHARED((NPAD, HHALF), jnp.float32),
        pltpu.VMEM((P1_CH, CHUNK), jnp.int32),
        pltpu.VMEM((P1_CH, CHUNK), jnp.int32),
        pltpu.VMEM((CHUNK, HHALF), jnp.float32),
        pltpu.VMEM((CHUNK, HHALF), jnp.float32),
        pltpu.SemaphoreType.DMA,
        pltpu.SemaphoreType.DMA,
    ],
)


def _prop2_body(ts_hbm, th_hbm, row_hbm, col_hbm, out_hbm,
                acc, row_v, col_v, buf0, buf1, sem0, sem1):
    c = lax.axis_index("c")
    s = lax.axis_index("s")
    pltpu.sync_copy(th_hbm.at[pl.ds(s * STRIPE, STRIPE)],
                    acc.at[pl.ds(s * STRIPE, STRIPE)])
    pltpu.sync_copy(row_hbm.at[c, s], row_v)
    pltpu.sync_copy(col_hbm.at[c, s], col_v)
    plsc.subcore_barrier()

    _pipe_group(ts_hbm, row_v, col_v, acc, buf0, buf1, sem0, sem1, P2_CH)
    plsc.subcore_barrier()
    pltpu.sync_copy(acc.at[pl.ds(s * STRIPE, STRIPE)],
                    out_hbm.at[c, pl.ds(s * STRIPE, STRIPE)])


_prop2_call = pl.kernel(
    _prop2_body,
    out_type=jax.ShapeDtypeStruct((NC, NPAD, TSW), jnp.float32),
    mesh=_MESH,
    scratch_types=[
        pltpu.VMEM_SHARED((NPAD, TSW), jnp.float32),
        pltpu.VMEM((P2_CH, CHUNK), jnp.int32),
        pltpu.VMEM((P2_CH, CHUNK), jnp.int32),
        pltpu.VMEM((CHUNK, TSW), jnp.float32),
        pltpu.VMEM((CHUNK, TSW), jnp.float32),
        pltpu.SemaphoreType.DMA,
        pltpu.SemaphoreType.DMA,
    ],
)


def _scale_body(x_ref, deg_ref, xs_ref, dinv_ref):
    dinv = lax.rsqrt(deg_ref[...])
    dinv_ref[...] = dinv
    xsb = x_ref[...] * dinv
    xs_ref[0] = xsb[:, :HHALF]
    xs_ref[1] = xsb[:, HHALF:]


_scale_call = pl.pallas_call(
    _scale_body,
    grid=(N // BM,),
    in_specs=[
        pl.BlockSpec((BM, D), lambda i: (i, 0)),
        pl.BlockSpec((BM, 1), lambda i: (i, 0)),
    ],
    out_specs=[
        pl.BlockSpec((NC, BM, HHALF), lambda i: (0, i, 0)),
        pl.BlockSpec((BM, 1), lambda i: (i, 0)),
    ],
    out_shape=[
        jax.ShapeDtypeStruct((NC, NPAD, HHALF), jnp.float32),
        jax.ShapeDtypeStruct((NPAD, 1), jnp.float32),
    ],
)


def _mid_body(p1_ref, dinv_ref, w1_ref, b1_ref, w2_ref, ts_ref, th_ref):
    dinv = dinv_ref[...]
    h0 = p1_ref[0] * dinv
    h1 = p1_ref[1] * dinv
    w1 = w1_ref[...]
    a = lax.dot_general(h0, w1[:, :HHALF], (((1,), (1,)), ((), ())),
                        preferred_element_type=jnp.float32)
    a = a + lax.dot_general(h1, w1[:, HHALF:], (((1,), (1,)), ((), ())),
                            preferred_element_type=jnp.float32)
    a = jnp.maximum(a + b1_ref[...], 0.0)
    t = lax.dot_general(a, w2_ref[...], (((1,), (1,)), ((), ())),
                        preferred_element_type=jnp.float32)
    ts = jnp.concatenate(
        [t * dinv, jnp.zeros((t.shape[0], TSW - C), jnp.float32)], axis=1)
    ts_ref[...] = ts
    th_ref[...] = 0.5 * ts


_mid_call = pl.pallas_call(
    _mid_body,
    grid=(N // BM,),
    in_specs=[
        pl.BlockSpec((NC, BM, HHALF), lambda i: (0, i, 0)),
        pl.BlockSpec((BM, 1), lambda i: (i, 0)),
        pl.BlockSpec((HID, D), lambda i: (0, 0)),
        pl.BlockSpec((1, HID), lambda i: (0, 0)),
        pl.BlockSpec((C, HID), lambda i: (0, 0)),
    ],
    out_specs=[
        pl.BlockSpec((BM, TSW), lambda i: (i, 0)),
        pl.BlockSpec((BM, TSW), lambda i: (i, 0)),
    ],
    out_shape=[
        jax.ShapeDtypeStruct((NPAD, TSW), jnp.float32),
        jax.ShapeDtypeStruct((NPAD, TSW), jnp.float32),
    ],
)


def _final_body(p2y_ref, p2z_ref, dy_ref, dz_ref, b2_ref,
                lsy_ref, dist_ref, lsz_ref):
    b2 = b2_ref[...]
    y = (p2y_ref[0] + p2y_ref[1])[:, :C] * dy_ref[...] + b2
    z = (p2z_ref[0] + p2z_ref[1])[:, :C] * dz_ref[...] + b2

    def logsm(v):
        m = jnp.max(v, axis=1, keepdims=True)
        return v - m - jnp.log(jnp.sum(jnp.exp(v - m), axis=1, keepdims=True))

    lsy_ref[...] = logsm(y)
    lsz_ref[...] = logsm(z)
    num = jnp.sum(y * z, axis=1, keepdims=True)
    den = jnp.maximum(
        jnp.sqrt(jnp.sum(y * y, axis=1, keepdims=True)
                 * jnp.sum(z * z, axis=1, keepdims=True)), 1e-8)
    dist_ref[...] = 1.0 - num / den


_final_call = pl.pallas_call(
    _final_body,
    grid=(N // BM,),
    in_specs=[
        pl.BlockSpec((NC, BM, TSW), lambda i: (0, i, 0)),
        pl.BlockSpec((NC, BM, TSW), lambda i: (0, i, 0)),
        pl.BlockSpec((BM, 1), lambda i: (i, 0)),
        pl.BlockSpec((BM, 1), lambda i: (i, 0)),
        pl.BlockSpec((1, C), lambda i: (0, 0)),
    ],
    out_specs=[
        pl.BlockSpec((BM, C), lambda i: (i, 0)),
        pl.BlockSpec((BM, 1), lambda i: (i, 0)),
        pl.BlockSpec((BM, C), lambda i: (i, 0)),
    ],
    out_shape=[
        jax.ShapeDtypeStruct((N, C), jnp.float32),
        jax.ShapeDtypeStruct((N, 1), jnp.float32),
        jax.ShapeDtypeStruct((N, C), jnp.float32),
    ],
)


def _one_graph(x, edge_index, deg, W1, b1_2d, W2):
    row, col = edge_index[0], edge_index[1]
    row1 = row.reshape(NS, P1_PH, P1_CH, CHUNK)
    col1 = col.reshape(NS, P1_PH, P1_CH, CHUNK)
    row2 = row.reshape(NC, NS, P2_CH, CHUNK)
    col2 = col.reshape(NC, NS, P2_CH, CHUNK)
    xs, dinv = _scale_call(x, deg)
    p1 = _prop1_call(xs, row1, col1)
    ts, th = _mid_call(p1, dinv, W1, b1_2d, W2)
    p2 = _prop2_call(ts, th, row2, col2)
    return p2, dinv


def kernel(x, edge_index, x_trans, edge_index_trans, W1, b1, W2, b2):
    ones = jnp.ones((NPAD,), jnp.float32)
    cols2 = jnp.stack([edge_index[1], edge_index_trans[1]])
    cols2 = cols2.reshape(NC, NS, DEG_CH, CHUNK)
    deg2 = _deg_call(cols2, ones)
    deg_y = deg2[0, 0, :, None]
    deg_z = deg2[1, 0, :, None]

    b1_2d = b1.reshape(1, HID)
    p2y, dinv_y = _one_graph(x, edge_index, deg_y, W1, b1_2d, W2)
    p2z, dinv_z = _one_graph(x_trans, edge_index_trans, deg_z, W1, b1_2d, W2)

    ls_y, dist, ls_z = _final_call(p2y, p2z, dinv_y, dinv_z,
                                   b2.reshape(1, C))
    dist = dist.reshape(N)
    return (ls_y, dist, ls_z, ls_y, ls_y)


# speedup vs baseline: 2.7747x; 1.0719x over previous
import functools

import jax
import jax.numpy as jnp
from jax import lax
from jax.experimental import pallas as pl
from jax.experimental.pallas import tpu as pltpu
from jax.experimental.pallas import tpu_sc as plsc

N = 10000
D = 256
HID = 256
C = 64
E = 160000

NC, NS = 2, 16
CHUNK = 125
HHALF = D // 2

DEG_CH = E // NS // CHUNK
P1_PH = 2
P1_CH = E // NS // P1_PH // CHUNK
P2_CH = E // NC // NS // CHUNK

NPAD = 10240
STRIPE = NPAD // NS

_MESH = plsc.VectorSubcoreMesh(
    core_axis_name="c", subcore_axis_name="s", num_cores=NC, num_subcores=NS)

BM = 1000


GROUP = 40


def _pipe_group(src, row_v, col_v, acc, buf0, buf1, sem0, sem1, nch):
    bufs = (buf0, buf1)
    sems = (sem0, sem1)

    def body(i, carry):
        base = i * GROUP
        ds = [pltpu.async_copy(src.at[row_v.at[base]], bufs[0], sems[0]),
              pltpu.async_copy(src.at[row_v.at[base + 1]], bufs[1], sems[1])]
        for k in range(GROUP):
            b = k % 2
            ds[b].wait()
            pltpu.sync_copy(bufs[b], acc.at[col_v.at[base + k]], add=True)
            if k + 2 < GROUP:
                ds[b] = pltpu.async_copy(src.at[row_v.at[base + k + 2]],
                                         bufs[b], sems[b])
        return carry

    lax.fori_loop(0, nch // GROUP, body, 0)


def _deg_body(col_hbm, ones_hbm, out_hbm, acc, col_v, ones_v):
    c = lax.axis_index("c")
    s = lax.axis_index("s")
    pltpu.sync_copy(ones_hbm.at[pl.ds(s * STRIPE, STRIPE)],
                    acc.at[pl.ds(s * STRIPE, STRIPE)])
    pltpu.sync_copy(ones_hbm.at[pl.ds(0, 128)], ones_v)
    pltpu.sync_copy(col_hbm.at[c, s], col_v)
    plsc.subcore_barrier()

    def body(j, carry):
        pltpu.sync_copy(ones_v.at[pl.ds(0, CHUNK)], acc.at[col_v.at[j]],
                        add=True)
        return carry

    lax.fori_loop(0, DEG_CH, body, 0)
    plsc.subcore_barrier()
    pltpu.sync_copy(acc.at[pl.ds(s * STRIPE, STRIPE)],
                    out_hbm.at[c, 0, pl.ds(s * STRIPE, STRIPE)])


_deg_call = pl.kernel(
    _deg_body,
    out_type=jax.ShapeDtypeStruct((NC, 1, NPAD), jnp.float32),
    mesh=_MESH,
    scratch_types=[
        pltpu.VMEM_SHARED((NPAD,), jnp.float32),
        pltpu.VMEM((DEG_CH, CHUNK), jnp.int32),
        pltpu.VMEM((128,), jnp.float32),
    ],
)


def _prop1_body(xs_hbm, row_hbm, col_hbm, out_hbm,
                acc, row_v, col_v, buf0, buf1, sem0, sem1):
    c = lax.axis_index("c")
    s = lax.axis_index("s")
    pltpu.sync_copy(xs_hbm.at[c, pl.ds(s * STRIPE, STRIPE)],
                    acc.at[pl.ds(s * STRIPE, STRIPE)])
    xs_c = xs_hbm.at[c]
    for p in range(P1_PH):
        pltpu.sync_copy(row_hbm.at[s, p], row_v)
        pltpu.sync_copy(col_hbm.at[s, p], col_v)
        if p == 0:
            plsc.subcore_barrier()
        _pipe_group(xs_c, row_v, col_v, acc, buf0, buf1, sem0, sem1,
                    P1_CH)
    plsc.subcore_barrier()
    pltpu.sync_copy(acc.at[pl.ds(s * STRIPE, STRIPE)],
                    out_hbm.at[c, pl.ds(s * STRIPE, STRIPE)])


_prop1_call = pl.kernel(
    _prop1_body,
    out_type=jax.ShapeDtypeStruct((NC, NPAD, HHALF), jnp.float32),
    mesh=_MESH,
    scratch_types=[
        pltpu.VMEM_SHARED((NPAD, HHALF), jnp.float32),
        pltpu.VMEM((P1_CH, CHUNK), jnp.int32),
        pltpu.VMEM((P1_CH, CHUNK), jnp.int32),
        pltpu.VMEM((CHUNK, HHALF), jnp.float32),
        pltpu.VMEM((CHUNK, HHALF), jnp.float32),
        pltpu.SemaphoreType.DMA,
        pltpu.SemaphoreType.DMA,
    ],
)


def _prop2_body(ts_hbm, th_hbm, row_hbm, col_hbm, out_hbm,
                acc, row_v, col_v, buf0, buf1, sem0, sem1):
    c = lax.axis_index("c")
    s = lax.axis_index("s")
    pltpu.sync_copy(th_hbm.at[pl.ds(s * STRIPE, STRIPE)],
                    acc.at[pl.ds(s * STRIPE, STRIPE)])
    pltpu.sync_copy(row_hbm.at[c, s], row_v)
    pltpu.sync_copy(col_hbm.at[c, s], col_v)
    plsc.subcore_barrier()

    _pipe_group(ts_hbm, row_v, col_v, acc, buf0, buf1, sem0, sem1, P2_CH)
    plsc.subcore_barrier()
    pltpu.sync_copy(acc.at[pl.ds(s * STRIPE, STRIPE)],
                    out_hbm.at[c, pl.ds(s * STRIPE, STRIPE)])


_prop2_call = pl.kernel(
    _prop2_body,
    out_type=jax.ShapeDtypeStruct((NC, NPAD, C), jnp.float32),
    mesh=_MESH,
    compiler_params=pltpu.CompilerParams(use_tc_tiling_on_sc=False),
    scratch_types=[
        pltpu.VMEM_SHARED((NPAD, C), jnp.float32),
        pltpu.VMEM((P2_CH, CHUNK), jnp.int32),
        pltpu.VMEM((P2_CH, CHUNK), jnp.int32),
        pltpu.VMEM((CHUNK, C), jnp.float32),
        pltpu.VMEM((CHUNK, C), jnp.float32),
        pltpu.SemaphoreType.DMA,
        pltpu.SemaphoreType.DMA,
    ],
)


def _scale_body(x_ref, deg_ref, xs_ref, dinv_ref):
    dinv = lax.rsqrt(deg_ref[...])
    dinv_ref[...] = dinv
    xsb = x_ref[...] * dinv
    xs_ref[0] = xsb[:, :HHALF]
    xs_ref[1] = xsb[:, HHALF:]


_scale_call = pl.pallas_call(
    _scale_body,
    grid=(N // BM,),
    in_specs=[
        pl.BlockSpec((BM, D), lambda i: (i, 0)),
        pl.BlockSpec((BM, 1), lambda i: (i, 0)),
    ],
    out_specs=[
        pl.BlockSpec((NC, BM, HHALF), lambda i: (0, i, 0)),
        pl.BlockSpec((BM, 1), lambda i: (i, 0)),
    ],
    out_shape=[
        jax.ShapeDtypeStruct((NC, NPAD, HHALF), jnp.float32),
        jax.ShapeDtypeStruct((NPAD, 1), jnp.float32),
    ],
)


def _mid_body(p1_ref, dinv_ref, w1_ref, b1_ref, w2_ref, ts_ref, th_ref):
    dinv = dinv_ref[...]
    h0 = p1_ref[0] * dinv
    h1 = p1_ref[1] * dinv
    w1 = w1_ref[...]
    a = lax.dot_general(h0, w1[:, :HHALF], (((1,), (1,)), ((), ())),
                        preferred_element_type=jnp.float32)
    a = a + lax.dot_general(h1, w1[:, HHALF:], (((1,), (1,)), ((), ())),
                            preferred_element_type=jnp.float32)
    a = jnp.maximum(a + b1_ref[...], 0.0)
    t = lax.dot_general(a, w2_ref[...], (((1,), (1,)), ((), ())),
                        preferred_element_type=jnp.float32)
    ts = t * dinv
    ts_ref[...] = ts
    th_ref[...] = 0.5 * ts


_mid_call = pl.pallas_call(
    _mid_body,
    grid=(N // BM,),
    in_specs=[
        pl.BlockSpec((NC, BM, HHALF), lambda i: (0, i, 0)),
        pl.BlockSpec((BM, 1), lambda i: (i, 0)),
        pl.BlockSpec((HID, D), lambda i: (0, 0)),
        pl.BlockSpec((1, HID), lambda i: (0, 0)),
        pl.BlockSpec((C, HID), lambda i: (0, 0)),
    ],
    out_specs=[
        pl.BlockSpec((BM, C), lambda i: (i, 0)),
        pl.BlockSpec((BM, C), lambda i: (i, 0)),
    ],
    out_shape=[
        jax.ShapeDtypeStruct((NPAD, C), jnp.float32),
        jax.ShapeDtypeStruct((NPAD, C), jnp.float32),
    ],
)


def _final_body(p2y_ref, p2z_ref, dy_ref, dz_ref, b2_ref,
                lsy_ref, dist_ref, lsz_ref):
    b2 = b2_ref[...]
    y = (p2y_ref[0] + p2y_ref[1]) * dy_ref[...] + b2
    z = (p2z_ref[0] + p2z_ref[1]) * dz_ref[...] + b2

    def logsm(v):
        m = jnp.max(v, axis=1, keepdims=True)
        return v - m - jnp.log(jnp.sum(jnp.exp(v - m), axis=1, keepdims=True))

    lsy_ref[...] = logsm(y)
    lsz_ref[...] = logsm(z)
    num = jnp.sum(y * z, axis=1, keepdims=True)
    den = jnp.maximum(
        jnp.sqrt(jnp.sum(y * y, axis=1, keepdims=True)
                 * jnp.sum(z * z, axis=1, keepdims=True)), 1e-8)
    dist_ref[...] = 1.0 - num / den


_final_call = pl.pallas_call(
    _final_body,
    grid=(N // BM,),
    in_specs=[
        pl.BlockSpec((NC, BM, C), lambda i: (0, i, 0)),
        pl.BlockSpec((NC, BM, C), lambda i: (0, i, 0)),
        pl.BlockSpec((BM, 1), lambda i: (i, 0)),
        pl.BlockSpec((BM, 1), lambda i: (i, 0)),
        pl.BlockSpec((1, C), lambda i: (0, 0)),
    ],
    out_specs=[
        pl.BlockSpec((BM, C), lambda i: (i, 0)),
        pl.BlockSpec((BM, 1), lambda i: (i, 0)),
        pl.BlockSpec((BM, C), lambda i: (i, 0)),
    ],
    out_shape=[
        jax.ShapeDtypeStruct((N, C), jnp.float32),
        jax.ShapeDtypeStruct((N, 1), jnp.float32),
        jax.ShapeDtypeStruct((N, C), jnp.float32),
    ],
)


def _one_graph(x, edge_index, deg, W1, b1_2d, W2):
    row, col = edge_index[0], edge_index[1]
    row1 = row.reshape(NS, P1_PH, P1_CH, CHUNK)
    col1 = col.reshape(NS, P1_PH, P1_CH, CHUNK)
    row2 = row.reshape(NC, NS, P2_CH, CHUNK)
    col2 = col.reshape(NC, NS, P2_CH, CHUNK)
    xs, dinv = _scale_call(x, deg)
    p1 = _prop1_call(xs, row1, col1)
    ts, th = _mid_call(p1, dinv, W1, b1_2d, W2)
    p2 = _prop2_call(ts, th, row2, col2)
    return p2, dinv


def kernel(x, edge_index, x_trans, edge_index_trans, W1, b1, W2, b2):
    ones = jnp.ones((NPAD,), jnp.float32)
    cols2 = jnp.stack([edge_index[1], edge_index_trans[1]])
    cols2 = cols2.reshape(NC, NS, DEG_CH, CHUNK)
    deg2 = _deg_call(cols2, ones)
    deg_y = deg2[0, 0, :, None]
    deg_z = deg2[1, 0, :, None]

    b1_2d = b1.reshape(1, HID)
    p2y, dinv_y = _one_graph(x, edge_index, deg_y, W1, b1_2d, W2)
    p2z, dinv_z = _one_graph(x_trans, edge_index_trans, deg_z, W1, b1_2d, W2)

    ls_y, dist, ls_z = _final_call(p2y, p2z, dinv_y, dinv_z,
                                   b2.reshape(1, C))
    dist = dist.reshape(N)
    return (ls_y, dist, ls_z, ls_y, ls_y)
